# Initial kernel scaffold; baseline (speedup 1.0000x reference)
import jax
import jax.numpy as jnp
from jax.experimental import pallas as pl


def kernel(x, table):
    raise NotImplementedError("write your pallas kernel here")


# speedup vs baseline: 4.1576x; 4.1576x over previous
import functools
import jax
import jax.numpy as jnp
from jax import lax
from jax.experimental import pallas as pl
from jax.experimental.pallas import tpu as pltpu
from jax.experimental.pallas import tpu_sc as plsc

NC = 2
NS = 16
NW = NC * NS

CHUNK = 512
IPG = 128
GPC = CHUNK // IPG

VOCAB = 1000
DIM = 64
NTOT = 16384 * 200


@functools.partial(
    pl.kernel,
    out_type=jax.ShapeDtypeStruct((NTOT, DIM), jnp.float32),
    mesh=plsc.VectorSubcoreMesh(core_axis_name="c", subcore_axis_name="s"),
    scratch_types=[
        pltpu.VMEM((CHUNK,), jnp.int32),
        pltpu.VMEM((CHUNK, DIM), jnp.float32),
        pltpu.SemaphoreType.DMA,
    ],
    compiler_params=pltpu.CompilerParams(use_tc_tiling_on_sc=False),
)
def _emb_lookup(x_hbm, table_hbm, out_hbm, idx_v, rows_v, sem):
    wid = lax.axis_index("s") * NC + lax.axis_index("c")
    n_per_w = NTOT // NW
    base = wid * n_per_w
    n_chunks = n_per_w // CHUNK

    def chunk_body(ci, carry):
        off = base + ci * CHUNK
        pltpu.sync_copy(x_hbm.at[pl.ds(off, CHUNK)], idx_v)
        copies = []
        for j in range(GPC):
            copies.append(
                pltpu.async_copy(
                    table_hbm.at[idx_v.at[pl.ds(j * IPG, IPG)]],
                    rows_v.at[pl.ds(j * IPG, IPG)],
                    sem,
                )
            )
        for c in copies:
            c.wait()
        pltpu.sync_copy(rows_v, out_hbm.at[pl.ds(off, CHUNK)])
        return carry

    lax.fori_loop(0, n_chunks, chunk_body, 0)


def kernel(x, table):
    xf = x.reshape(-1).astype(jnp.int32)
    out = _emb_lookup(xf, table)
    return out.reshape(x.shape + (table.shape[1],))


# speedup vs baseline: 4.1604x; 1.0007x over previous
import functools
import jax
import jax.numpy as jnp
from jax import lax
from jax.experimental import pallas as pl
from jax.experimental.pallas import tpu as pltpu
from jax.experimental.pallas import tpu_sc as plsc

NC = 2
NS = 16
NW = NC * NS

CHUNK = 512
IPG = 128
GPC = CHUNK // IPG

VOCAB = 1000
DIM = 64
NTOT = 16384 * 200


@functools.partial(
    pl.kernel,
    out_type=jax.ShapeDtypeStruct((NTOT, DIM), jnp.float32),
    mesh=plsc.VectorSubcoreMesh(core_axis_name="c", subcore_axis_name="s"),
    scratch_types=[
        pltpu.VMEM((2, CHUNK), jnp.int32),
        pltpu.VMEM((2, CHUNK, DIM), jnp.float32),
        pltpu.SemaphoreType.DMA,
        pltpu.SemaphoreType.DMA,
        pltpu.SemaphoreType.DMA,
    ],
    compiler_params=pltpu.CompilerParams(use_tc_tiling_on_sc=False),
)
def _emb_lookup(x_hbm, table_hbm, out_hbm, idx_v, rows_v, sem_i, sem_g, sem_o):
    wid = lax.axis_index("s") * NC + lax.axis_index("c")
    n_per_w = NTOT // NW
    base = wid * n_per_w
    n_chunks = n_per_w // CHUNK

    def fire_idx(ci, b):
        return pltpu.async_copy(
            x_hbm.at[pl.ds(base + ci * CHUNK, CHUNK)], idx_v.at[b], sem_i
        )

    def fire_gathers(b):
        for j in range(GPC):
            pltpu.async_copy(
                table_hbm.at[idx_v.at[b].at[pl.ds(j * IPG, IPG)]],
                rows_v.at[b].at[pl.ds(j * IPG, IPG)],
                sem_g,
            )

    def wait_gathers(b):
        pltpu.make_async_copy(
            table_hbm.at[idx_v.at[b]], rows_v.at[b], sem_g
        ).wait()

    def fire_out(ci, b):
        return pltpu.async_copy(
            rows_v.at[b], out_hbm.at[pl.ds(base + ci * CHUNK, CHUNK)], sem_o
        )

    def wait_out(ci, b):
        pltpu.make_async_copy(
            rows_v.at[b], out_hbm.at[pl.ds(base + ci * CHUNK, CHUNK)], sem_o
        ).wait()

    fire_idx(0, 0).wait()
    fire_gathers(0)
    fire_idx(1, 1).wait()
    wait_gathers(0)
    fire_out(0, 0)
    fire_gathers(1)

    def body(ci, carry):
        b = ci % 2
        fire_idx(ci, b).wait()
        wait_out(ci - 2, b)
        wait_gathers(1 - b)
        fire_out(ci - 1, 1 - b)
        fire_gathers(b)
        return carry

    lax.fori_loop(2, n_chunks, body, 0)

    last = n_chunks - 1
    wait_out(last - 1, (last - 1) % 2)
    wait_gathers(last % 2)
    fire_out(last, last % 2)
    wait_out(last, last % 2)


def kernel(x, table):
    xf = x.reshape(-1).astype(jnp.int32)
    out = _emb_lookup(xf, table)
    return out.reshape(x.shape + (table.shape[1],))
